# baseline (speedup 1.0000x reference)
import dataclasses
import functools

import jax
import jax.numpy as jnp
from jax import lax
from jax.experimental import pallas as pl
from jax.experimental.pallas import tpu as pltpu
from jax.experimental.pallas import tpu_sc as plsc

N = 10000
E = 320000
D = 128
H = 128

NC = 2
NS = 16
NW = NC * NS

CH = 64
NCHUNK = 156
CHT = 16
EPT = E // NW

N_PAD = 10240
RPT = N_PAD // NS

_mesh = plsc.VectorSubcoreMesh(core_axis_name="c", subcore_axis_name="s")


def _sc_aggregate(with_counts):
    out_type = [jax.ShapeDtypeStruct((NC * N_PAD, D), jnp.float32)]
    if with_counts:
        out_type.append(jax.ShapeDtypeStruct((NW, N_PAD), jnp.float32))
    scratch = (
        [pltpu.VMEM((1, CH), jnp.int32)] * 4 +
        [pltpu.VMEM((1, CH), jnp.int32)] * 4 +
        [pltpu.VMEM((CH, D), jnp.float32)] * 4 +
        [pltpu.VMEM((1, CHT), jnp.int32)] * 2 +
        [pltpu.VMEM((CHT, D), jnp.float32)] +
        [pltpu.VMEM_SHARED((N_PAD, D), jnp.float32)] +
        [pltpu.SemaphoreType.DMA] * 12
    )
    if with_counts:
        scratch.append(pltpu.VMEM((N_PAD,), jnp.float32))
        cp = pltpu.CompilerParams()
        if "needs_layout_passes" in pltpu.CompilerParams.__dataclass_fields__:
            cp = dataclasses.replace(cp, needs_layout_passes=False)
    else:
        cp = None

    @functools.partial(pl.kernel, out_type=out_type, mesh=_mesh,
                       scratch_types=scratch, compiler_params=cp)
    def body(x_hbm, src_hbm, dst_hbm, srct_hbm, dstt_hbm, zf_hbm, zc_hbm,
             *rest):
        if with_counts:
            sum_out, cnt_out = rest[:2]
            rr = rest[2:]
            hist = rr[28]
        else:
            sum_out = rest[0]
            rr = rest[1:]
        srcs = list(rr[0:4])
        dsts = list(rr[4:8])
        rows = list(rr[8:12])
        src_t, dst_t, rows_t = rr[12:15]
        accf = rr[15]
        gsem = list(rr[16:20])
        ssem = list(rr[20:24])
        isem = list(rr[24:28])
        rows0 = rows[0]
        c = lax.axis_index("c")
        s = lax.axis_index("s")
        wid = s * NC + c
        r0 = s * RPT

        if with_counts:
            pltpu.sync_copy(zc_hbm, hist)

        pltpu.sync_copy(zf_hbm, rows0)

        @pl.loop(0, RPT // CH)
        def _(j):
            pltpu.sync_copy(rows0, accf.at[pl.ds(r0 + j * CH, CH)])

        plsc.subcore_barrier()

        def _hist_update(dst_b):
            if with_counts:
                for j in range(CH // 16):
                    idx16 = dst_b[0, pl.ds(j * 16, 16)]
                    plsc.addupdate_scatter(
                        hist, [idx16], jnp.full((16,), 1.0, jnp.float32))

        base = wid * NCHUNK

        def _wait_scatter(b):
            pltpu.make_async_copy(
                rows[b], accf.at[dsts[b].at[0]], ssem[b]).wait()

        def _half(k, P):
            Q = (P + 1) % 4
            Rb = (P + 2) % 4

            @pl.when(k + 1 < NCHUNK)
            def _():
                pltpu.make_async_copy(
                    src_hbm.at[base + k + 1], srcs[Q], isem[Q]).wait()
                pltpu.make_async_copy(
                    dst_hbm.at[base + k + 1], dsts[Q], isem[Q]).wait()
                pltpu.async_copy(x_hbm.at[srcs[Q].at[0]], rows[Q], gsem[Q])

            pltpu.make_async_copy(
                x_hbm.at[srcs[P].at[0]], rows[P], gsem[P]).wait()
            pltpu.async_copy(rows[P], accf.at[dsts[P].at[0]], ssem[P],
                             add=True)
            _hist_update(dsts[P])

            @pl.when(k + 2 < NCHUNK)
            def _():
                @pl.when(k >= 2)
                def _():
                    _wait_scatter(Rb)
                pltpu.async_copy(src_hbm.at[base + k + 2], srcs[Rb],
                                 isem[Rb])
                pltpu.async_copy(dst_hbm.at[base + k + 2], dsts[Rb],
                                 isem[Rb])

        pltpu.sync_copy(src_hbm.at[base], srcs[0])
        pltpu.sync_copy(dst_hbm.at[base], dsts[0])
        pltpu.async_copy(x_hbm.at[srcs[0].at[0]], rows[0], gsem[0])
        pltpu.async_copy(src_hbm.at[base + 1], srcs[1], isem[1])
        pltpu.async_copy(dst_hbm.at[base + 1], dsts[1], isem[1])

        @pl.loop(0, NCHUNK, step=4)
        def _(k):
            _half(k, 0)
            _half(k + 1, 1)
            _half(k + 2, 2)
            _half(k + 3, 3)

        pltpu.sync_copy(srct_hbm.at[wid], src_t)
        pltpu.sync_copy(dstt_hbm.at[wid], dst_t)
        pltpu.sync_copy(x_hbm.at[src_t.at[0]], rows_t)
        pltpu.sync_copy(rows_t, accf.at[dst_t.at[0]], add=True)
        if with_counts:
            plsc.addupdate_scatter(hist, [dst_t[0, pl.ds(0, CHT)]],
                                   jnp.full((CHT,), 1.0, jnp.float32))

        _wait_scatter((NCHUNK - 4) % 4)
        _wait_scatter((NCHUNK - 3) % 4)
        _wait_scatter((NCHUNK - 2) % 4)
        _wait_scatter((NCHUNK - 1) % 4)

        plsc.subcore_barrier()

        o0 = c * N_PAD + r0

        @pl.loop(0, RPT // CH)
        def _(j):
            pltpu.sync_copy(accf.at[pl.ds(r0 + j * CH, CH)], rows0)
            pltpu.sync_copy(rows0, sum_out.at[pl.ds(o0 + j * CH, CH)])

        if with_counts:
            pltpu.sync_copy(hist, cnt_out.at[wid])

    return body


_sc_agg_counts = _sc_aggregate(True)
_sc_agg = _sc_aggregate(False)


R = 1000
NBLK = N // R


def _dot_t(a, w):
    return lax.dot_general(a, w, (((1,), (1,)), ((), ())),
                           preferred_element_type=jnp.float32,
                           precision=lax.Precision.HIGHEST)


def _mean(sumr, cntblk):
    summed = sumr[0] + sumr[1]
    ones = jnp.ones((NW, 1), jnp.float32)
    cnt = lax.dot_general(cntblk, ones, (((1,), (0,)), ((), ())),
                          preferred_element_type=jnp.float32,
                          precision=lax.Precision.HIGHEST)
    return summed * (1.0 / jnp.maximum(cnt, 1.0))


def _tcr_body(a, w, b, out):
    out[...] = _dot_t(a[...], w[...]) + b[...]


def _tc1_body(sumr, cntr, xr, wl, h1_out):
    mean = _mean(sumr[...], cntr[...])
    h1_out[...] = jnp.maximum(_dot_t(mean, wl[...]) + xr[...], 0.0)


def _tc2_body(sumr, cntr, hr, wl, wi, bi_bh, wo, bo, sig_out, hid_out):
    mean = _mean(sumr[...], cntr[...])
    h2 = _dot_t(mean, wl[...]) + hr[...]
    hidden = jnp.tanh(_dot_t(h2, wi[...]) + bi_bh[...])
    hid_out[...] = hidden
    sig_out[...] = jax.nn.sigmoid(_dot_t(hidden, wo[...]) + bo[...])


_row_spec = pl.BlockSpec((R, D), lambda i: (i, 0))
_sum_spec = pl.BlockSpec((2, R, D), lambda i: (0, i, 0))
_cnt_spec = pl.BlockSpec((R, NW), lambda i: (i, 0))
_w_spec = pl.BlockSpec((H, D), lambda i: (0, 0))
_b_spec = pl.BlockSpec((1, H), lambda i: (0, 0))


def kernel(x, edge_index, W_l1, W_r1, b1, W_l2, W_r2, b2, Wi, bi, Wh, bh, Wo, bo):
    src2 = edge_index[0].reshape(NW, EPT)
    dst2 = edge_index[1].reshape(NW, EPT)
    src = src2[:, :NCHUNK * CH].reshape(NW * NCHUNK, 1, CH)
    dst = dst2[:, :NCHUNK * CH].reshape(NW * NCHUNK, 1, CH)
    src_t = src2[:, NCHUNK * CH:].reshape(NW, 1, CHT)
    dst_t = dst2[:, NCHUNK * CH:].reshape(NW, 1, CHT)
    zf = jnp.zeros((CH, D), jnp.float32)
    zc = jnp.zeros((N_PAD,), jnp.float32)

    def _lin_r(a, w, b):
        return pl.pallas_call(
            _tcr_body,
            grid=(NBLK,),
            in_specs=[_row_spec, _w_spec, _b_spec],
            out_specs=_row_spec,
            out_shape=jax.ShapeDtypeStruct((N, H), jnp.float32),
        )(a, w, b.reshape(1, H))

    xr = _lin_r(x, W_r1, b1)
    sum1, cnt = _sc_agg_counts(x, src, dst, src_t, dst_t, zf, zc)
    sum1 = sum1.reshape(NC, N_PAD, D)
    cnt = cnt.T

    h1 = pl.pallas_call(
        _tc1_body,
        grid=(NBLK,),
        in_specs=[_sum_spec, _cnt_spec, _row_spec, _w_spec],
        out_specs=_row_spec,
        out_shape=jax.ShapeDtypeStruct((N, H), jnp.float32),
    )(sum1, cnt, xr, W_l1)

    hr = _lin_r(h1, W_r2, b2)
    (sum2,) = _sc_agg(h1, src, dst, src_t, dst_t, zf, zc)
    sum2 = sum2.reshape(NC, N_PAD, D)

    sig, hidden = pl.pallas_call(
        _tc2_body,
        grid=(NBLK,),
        in_specs=[_sum_spec, _cnt_spec, _row_spec, _w_spec,
                  _w_spec, _b_spec, _w_spec, _b_spec],
        out_specs=[_row_spec, _row_spec],
        out_shape=[jax.ShapeDtypeStruct((N, H), jnp.float32),
                   jax.ShapeDtypeStruct((N, H), jnp.float32)],
    )(sum2, cnt, hr, W_l2, Wi,
      (bi + bh).reshape(1, H), Wo, bo.reshape(1, H))

    return (sig, hidden)


# speedup vs baseline: 1.0456x; 1.0456x over previous
import dataclasses
import functools

import jax
import jax.numpy as jnp
from jax import lax
from jax.experimental import pallas as pl
from jax.experimental.pallas import tpu as pltpu
from jax.experimental.pallas import tpu_sc as plsc

N = 10000
E = 320000
D = 128
H = 128

NC = 2
NS = 16
NW = NC * NS

CH = 64
NCHUNK = 156
CHT = 16
EPT = E // NW

N_PAD = 10240
RPT = N_PAD // NS

_mesh = plsc.VectorSubcoreMesh(core_axis_name="c", subcore_axis_name="s")


def _sc_aggregate(with_counts):
    out_type = [jax.ShapeDtypeStruct((NC * N_PAD, D), jnp.float32)]
    if with_counts:
        out_type.append(jax.ShapeDtypeStruct((NW, N_PAD), jnp.float32))
    scratch = (
        [pltpu.VMEM((1, CH), jnp.int32)] * 4 +
        [pltpu.VMEM((1, CH), jnp.int32)] * 4 +
        [pltpu.VMEM((CH, D), jnp.float32)] * 4 +
        [pltpu.VMEM((1, CHT), jnp.int32)] * 2 +
        [pltpu.VMEM((CHT, D), jnp.float32)] +
        [pltpu.VMEM_SHARED((N_PAD, D), jnp.float32)] +
        [pltpu.SemaphoreType.DMA] * 12
    )
    if with_counts:
        scratch.append(pltpu.VMEM((N_PAD,), jnp.float32))
        cp = pltpu.CompilerParams()
        if "needs_layout_passes" in pltpu.CompilerParams.__dataclass_fields__:
            cp = dataclasses.replace(cp, needs_layout_passes=False)
    else:
        cp = None

    @functools.partial(pl.kernel, out_type=out_type, mesh=_mesh,
                       scratch_types=scratch, compiler_params=cp)
    def body(x_hbm, src_hbm, dst_hbm, srct_hbm, dstt_hbm, zf_hbm, zc_hbm,
             *rest):
        if with_counts:
            sum_out, cnt_out = rest[:2]
            rr = rest[2:]
            hist = rr[28]
        else:
            sum_out = rest[0]
            rr = rest[1:]
        srcs = list(rr[0:4])
        dsts = list(rr[4:8])
        rows = list(rr[8:12])
        src_t, dst_t, rows_t = rr[12:15]
        accf = rr[15]
        gsem = list(rr[16:20])
        ssem = list(rr[20:24])
        isem = list(rr[24:28])
        rows0 = rows[0]
        c = lax.axis_index("c")
        s = lax.axis_index("s")
        wid = s * NC + c
        r0 = s * RPT

        if with_counts:
            pltpu.sync_copy(zc_hbm, hist)

        pltpu.sync_copy(zf_hbm, rows0)

        @pl.loop(0, RPT // CH)
        def _(j):
            pltpu.async_copy(rows0, accf.at[pl.ds(r0 + j * CH, CH)],
                             gsem[0])

        @pl.loop(0, RPT // CH)
        def _(j):
            pltpu.make_async_copy(
                rows0, accf.at[pl.ds(r0, CH)], gsem[0]).wait()

        plsc.subcore_barrier()

        def _hist_update(dst_b):
            if with_counts:
                for j in range(CH // 16):
                    idx16 = dst_b[0, pl.ds(j * 16, 16)]
                    plsc.addupdate_scatter(
                        hist, [idx16], jnp.full((16,), 1.0, jnp.float32))

        base = wid * NCHUNK

        def _wait_scatter(b):
            pltpu.make_async_copy(
                rows[b], accf.at[dsts[b].at[0]], ssem[b]).wait()

        def _half(k, P):
            Q = (P + 1) % 4
            Rb = (P + 2) % 4

            @pl.when(k + 1 < NCHUNK)
            def _():
                pltpu.make_async_copy(
                    src_hbm.at[base + k + 1], srcs[Q], isem[Q]).wait()
                pltpu.make_async_copy(
                    dst_hbm.at[base + k + 1], dsts[Q], isem[Q]).wait()
                pltpu.async_copy(x_hbm.at[srcs[Q].at[0]], rows[Q], gsem[Q])

            pltpu.make_async_copy(
                x_hbm.at[srcs[P].at[0]], rows[P], gsem[P]).wait()
            pltpu.async_copy(rows[P], accf.at[dsts[P].at[0]], ssem[P],
                             add=True)
            _hist_update(dsts[P])

            @pl.when(k + 2 < NCHUNK)
            def _():
                @pl.when(k >= 2)
                def _():
                    _wait_scatter(Rb)
                pltpu.async_copy(src_hbm.at[base + k + 2], srcs[Rb],
                                 isem[Rb])
                pltpu.async_copy(dst_hbm.at[base + k + 2], dsts[Rb],
                                 isem[Rb])

        pltpu.sync_copy(src_hbm.at[base], srcs[0])
        pltpu.sync_copy(dst_hbm.at[base], dsts[0])
        pltpu.async_copy(x_hbm.at[srcs[0].at[0]], rows[0], gsem[0])
        pltpu.async_copy(src_hbm.at[base + 1], srcs[1], isem[1])
        pltpu.async_copy(dst_hbm.at[base + 1], dsts[1], isem[1])

        @pl.loop(0, NCHUNK, step=4)
        def _(k):
            _half(k, 0)
            _half(k + 1, 1)
            _half(k + 2, 2)
            _half(k + 3, 3)

        pltpu.sync_copy(srct_hbm.at[wid], src_t)
        pltpu.sync_copy(dstt_hbm.at[wid], dst_t)
        pltpu.sync_copy(x_hbm.at[src_t.at[0]], rows_t)
        pltpu.sync_copy(rows_t, accf.at[dst_t.at[0]], add=True)
        if with_counts:
            plsc.addupdate_scatter(hist, [dst_t[0, pl.ds(0, CHT)]],
                                   jnp.full((CHT,), 1.0, jnp.float32))

        _wait_scatter((NCHUNK - 4) % 4)
        _wait_scatter((NCHUNK - 3) % 4)
        _wait_scatter((NCHUNK - 2) % 4)
        _wait_scatter((NCHUNK - 1) % 4)

        plsc.subcore_barrier()

        o0 = c * N_PAD + r0

        rows1 = rows[1]

        @pl.loop(0, RPT // CH, step=2)
        def _(j):
            @pl.when(j >= 2)
            def _():
                pltpu.make_async_copy(
                    rows0, sum_out.at[pl.ds(o0, CH)], gsem[0]).wait()
                pltpu.make_async_copy(
                    rows1, sum_out.at[pl.ds(o0, CH)], gsem[1]).wait()
            pltpu.sync_copy(accf.at[pl.ds(r0 + j * CH, CH)], rows0)
            pltpu.async_copy(rows0, sum_out.at[pl.ds(o0 + j * CH, CH)],
                             gsem[0])
            pltpu.sync_copy(accf.at[pl.ds(r0 + (j + 1) * CH, CH)], rows1)
            pltpu.async_copy(rows1, sum_out.at[pl.ds(o0 + (j + 1) * CH, CH)],
                             gsem[1])

        pltpu.make_async_copy(
            rows0, sum_out.at[pl.ds(o0, CH)], gsem[0]).wait()
        pltpu.make_async_copy(
            rows1, sum_out.at[pl.ds(o0, CH)], gsem[1]).wait()

        if with_counts:
            pltpu.sync_copy(hist, cnt_out.at[wid])

    return body


_sc_agg_counts = _sc_aggregate(True)
_sc_agg = _sc_aggregate(False)


R = 1000
NBLK = N // R


def _dot_t(a, w):
    return lax.dot_general(a, w, (((1,), (1,)), ((), ())),
                           preferred_element_type=jnp.float32,
                           precision=lax.Precision.HIGHEST)


def _mean(sumr, cntblk):
    summed = sumr[0] + sumr[1]
    ones = jnp.ones((NW, 1), jnp.float32)
    cnt = lax.dot_general(cntblk, ones, (((1,), (0,)), ((), ())),
                          preferred_element_type=jnp.float32,
                          precision=lax.Precision.HIGHEST)
    return summed * (1.0 / jnp.maximum(cnt, 1.0))


def _tc1_body(sumr, cntr, x, wl, wr, b, h1_out):
    mean = _mean(sumr[...], cntr[...])
    h1_out[...] = jnp.maximum(
        _dot_t(mean, wl[...]) + _dot_t(x[...], wr[...]) + b[...], 0.0)


def _tc2_body(sumr, cntr, h1, wl, wr, b, wi, bi_bh, wo, bo, sig_out, hid_out):
    mean = _mean(sumr[...], cntr[...])
    h2 = _dot_t(mean, wl[...]) + _dot_t(h1[...], wr[...]) + b[...]
    hidden = jnp.tanh(_dot_t(h2, wi[...]) + bi_bh[...])
    hid_out[...] = hidden
    sig_out[...] = jax.nn.sigmoid(_dot_t(hidden, wo[...]) + bo[...])


_row_spec = pl.BlockSpec((R, D), lambda i: (i, 0))
_sum_spec = pl.BlockSpec((2, R, D), lambda i: (0, i, 0))
_cnt_spec = pl.BlockSpec((R, NW), lambda i: (i, 0))
_w_spec = pl.BlockSpec((H, D), lambda i: (0, 0))
_b_spec = pl.BlockSpec((1, H), lambda i: (0, 0))


def kernel(x, edge_index, W_l1, W_r1, b1, W_l2, W_r2, b2, Wi, bi, Wh, bh, Wo, bo):
    src2 = edge_index[0].reshape(NW, EPT)
    dst2 = edge_index[1].reshape(NW, EPT)
    src = src2[:, :NCHUNK * CH].reshape(NW * NCHUNK, 1, CH)
    dst = dst2[:, :NCHUNK * CH].reshape(NW * NCHUNK, 1, CH)
    src_t = src2[:, NCHUNK * CH:].reshape(NW, 1, CHT)
    dst_t = dst2[:, NCHUNK * CH:].reshape(NW, 1, CHT)
    zf = jnp.zeros((CH, D), jnp.float32)
    zc = jnp.zeros((N_PAD,), jnp.float32)

    sum1, cnt = _sc_agg_counts(x, src, dst, src_t, dst_t, zf, zc)
    sum1 = sum1.reshape(NC, N_PAD, D)
    cnt = cnt.T

    b1r = b1.reshape(1, H)
    h1 = pl.pallas_call(
        _tc1_body,
        grid=(NBLK,),
        in_specs=[_sum_spec, _cnt_spec, _row_spec, _w_spec, _w_spec, _b_spec],
        out_specs=_row_spec,
        out_shape=jax.ShapeDtypeStruct((N, H), jnp.float32),
    )(sum1, cnt, x, W_l1, W_r1, b1r)

    (sum2,) = _sc_agg(h1, src, dst, src_t, dst_t, zf, zc)
    sum2 = sum2.reshape(NC, N_PAD, D)

    sig, hidden = pl.pallas_call(
        _tc2_body,
        grid=(NBLK,),
        in_specs=[_sum_spec, _cnt_spec, _row_spec, _w_spec, _w_spec, _b_spec,
                  _w_spec, _b_spec, _w_spec, _b_spec],
        out_specs=[_row_spec, _row_spec],
        out_shape=[jax.ShapeDtypeStruct((N, H), jnp.float32),
                   jax.ShapeDtypeStruct((N, H), jnp.float32)],
    )(sum2, cnt, h1, W_l2, W_r2, b2.reshape(1, H), Wi,
      (bi + bh).reshape(1, H), Wo, bo.reshape(1, H))

    return (sig, hidden)


# speedup vs baseline: 1.2111x; 1.1584x over previous
import dataclasses
import functools

import jax
import jax.numpy as jnp
from jax import lax
from jax.experimental import pallas as pl
from jax.experimental.pallas import tpu as pltpu
from jax.experimental.pallas import tpu_sc as plsc

N = 10000
E = 320000
D = 128
H = 128

NC = 2
NS = 16
NW = NC * NS

CH = 64
NCHUNK = 156
CHT = 16
EPT = E // NW

N_PAD = 10240
RPT = N_PAD // NS

_mesh = plsc.VectorSubcoreMesh(core_axis_name="c", subcore_axis_name="s")


def _sc_aggregate(with_counts):
    out_type = [jax.ShapeDtypeStruct((NC * N_PAD, D), jnp.float32)]
    if with_counts:
        out_type.append(jax.ShapeDtypeStruct((NW, N_PAD), jnp.float32))
    scratch = (
        [pltpu.VMEM((1, CH), jnp.int32)] * 4 +
        [pltpu.VMEM((1, CH), jnp.int32)] * 4 +
        [pltpu.VMEM((CH, D), jnp.float32)] * 4 +
        [pltpu.VMEM((1, CHT), jnp.int32)] * 2 +
        [pltpu.VMEM((CHT, D), jnp.float32)] +
        [pltpu.VMEM_SHARED((N_PAD, D), jnp.float32)] +
        [pltpu.SemaphoreType.DMA] * 12
    )
    if with_counts:
        scratch.append(pltpu.VMEM((N_PAD,), jnp.float32))
        cp = pltpu.CompilerParams()
        if "needs_layout_passes" in pltpu.CompilerParams.__dataclass_fields__:
            cp = dataclasses.replace(cp, needs_layout_passes=False)
    else:
        cp = None

    @functools.partial(pl.kernel, out_type=out_type, mesh=_mesh,
                       scratch_types=scratch, compiler_params=cp)
    def body(x_hbm, src_hbm, dst_hbm, srct_hbm, dstt_hbm, zf_hbm, zc_hbm,
             *rest):
        if with_counts:
            sum_out, cnt_out = rest[:2]
            rr = rest[2:]
            hist = rr[28]
        else:
            sum_out = rest[0]
            rr = rest[1:]
        srcs = list(rr[0:4])
        dsts = list(rr[4:8])
        rows = list(rr[8:12])
        src_t, dst_t, rows_t = rr[12:15]
        accf = rr[15]
        gsem = list(rr[16:20])
        ssem = list(rr[20:24])
        isem = list(rr[24:28])
        rows0 = rows[0]
        c = lax.axis_index("c")
        s = lax.axis_index("s")
        wid = s * NC + c
        r0 = s * RPT

        if with_counts:
            pltpu.sync_copy(zc_hbm, hist)

        pltpu.sync_copy(zf_hbm, rows0)

        @pl.loop(0, RPT // CH)
        def _(j):
            pltpu.async_copy(rows0, accf.at[pl.ds(r0 + j * CH, CH)],
                             gsem[0])

        @pl.loop(0, RPT // CH)
        def _(j):
            pltpu.make_async_copy(
                rows0, accf.at[pl.ds(r0, CH)], gsem[0]).wait()

        plsc.subcore_barrier()

        def _hist_update(dst_b):
            if with_counts:
                for j in range(CH // 16):
                    idx16 = dst_b[0, pl.ds(j * 16, 16)]
                    plsc.addupdate_scatter(
                        hist, [idx16], jnp.full((16,), 1.0, jnp.float32))

        base = wid * NCHUNK

        def _wait_scatter(b):
            pltpu.make_async_copy(
                rows[b], accf.at[dsts[b].at[0]], ssem[b]).wait()

        def _half(k, P):
            Q = (P + 1) % 4
            Rb = (P + 2) % 4

            @pl.when(k + 1 < NCHUNK)
            def _():
                pltpu.make_async_copy(
                    src_hbm.at[base + k + 1], srcs[Q], isem[Q]).wait()
                pltpu.make_async_copy(
                    dst_hbm.at[base + k + 1], dsts[Q], isem[Q]).wait()
                pltpu.async_copy(x_hbm.at[srcs[Q].at[0]], rows[Q], gsem[Q])

            pltpu.make_async_copy(
                x_hbm.at[srcs[P].at[0]], rows[P], gsem[P]).wait()
            pltpu.async_copy(rows[P], accf.at[dsts[P].at[0]], ssem[P],
                             add=True)
            _hist_update(dsts[P])

            @pl.when(k + 2 < NCHUNK)
            def _():
                @pl.when(k >= 2)
                def _():
                    _wait_scatter(Rb)
                pltpu.async_copy(src_hbm.at[base + k + 2], srcs[Rb],
                                 isem[Rb])
                pltpu.async_copy(dst_hbm.at[base + k + 2], dsts[Rb],
                                 isem[Rb])

        pltpu.sync_copy(src_hbm.at[base], srcs[0])
        pltpu.sync_copy(dst_hbm.at[base], dsts[0])
        pltpu.async_copy(x_hbm.at[srcs[0].at[0]], rows[0], gsem[0])
        pltpu.async_copy(src_hbm.at[base + 1], srcs[1], isem[1])
        pltpu.async_copy(dst_hbm.at[base + 1], dsts[1], isem[1])

        @pl.loop(0, NCHUNK, step=4)
        def _(k):
            _half(k, 0)
            _half(k + 1, 1)
            _half(k + 2, 2)
            _half(k + 3, 3)

        pltpu.sync_copy(srct_hbm.at[wid], src_t)
        pltpu.sync_copy(dstt_hbm.at[wid], dst_t)
        pltpu.sync_copy(x_hbm.at[src_t.at[0]], rows_t)
        pltpu.sync_copy(rows_t, accf.at[dst_t.at[0]], add=True)
        if with_counts:
            plsc.addupdate_scatter(hist, [dst_t[0, pl.ds(0, CHT)]],
                                   jnp.full((CHT,), 1.0, jnp.float32))

        _wait_scatter((NCHUNK - 4) % 4)
        _wait_scatter((NCHUNK - 3) % 4)
        _wait_scatter((NCHUNK - 2) % 4)
        _wait_scatter((NCHUNK - 1) % 4)

        plsc.subcore_barrier()

        o0 = c * N_PAD + r0

        rows1 = rows[1]

        @pl.loop(0, RPT // CH, step=2)
        def _(j):
            @pl.when(j >= 2)
            def _():
                pltpu.make_async_copy(
                    rows0, sum_out.at[pl.ds(o0, CH)], gsem[0]).wait()
                pltpu.make_async_copy(
                    rows1, sum_out.at[pl.ds(o0, CH)], gsem[1]).wait()
            pltpu.sync_copy(accf.at[pl.ds(r0 + j * CH, CH)], rows0)
            pltpu.async_copy(rows0, sum_out.at[pl.ds(o0 + j * CH, CH)],
                             gsem[0])
            pltpu.sync_copy(accf.at[pl.ds(r0 + (j + 1) * CH, CH)], rows1)
            pltpu.async_copy(rows1, sum_out.at[pl.ds(o0 + (j + 1) * CH, CH)],
                             gsem[1])

        pltpu.make_async_copy(
            rows0, sum_out.at[pl.ds(o0, CH)], gsem[0]).wait()
        pltpu.make_async_copy(
            rows1, sum_out.at[pl.ds(o0, CH)], gsem[1]).wait()

        if with_counts:
            pltpu.sync_copy(hist, cnt_out.at[wid])

    return body


_sc_agg_counts = _sc_aggregate(True)
_sc_agg = _sc_aggregate(False)


R = 1000
NBLK = N // R


def _dot_t(a, w):
    return lax.dot_general(a, w, (((1,), (1,)), ((), ())),
                           preferred_element_type=jnp.float32,
                           precision=lax.Precision.DEFAULT)


def _mean(sumr, cntblk):
    summed = sumr[0] + sumr[1]
    ones = jnp.ones((NW, 1), jnp.float32)
    cnt = lax.dot_general(cntblk, ones, (((1,), (0,)), ((), ())),
                          preferred_element_type=jnp.float32,
                          precision=lax.Precision.DEFAULT)
    return summed * (1.0 / jnp.maximum(cnt, 1.0))


def _tc1_body(sumr, cntr, x, wl, wr, b, h1_out):
    mean = _mean(sumr[...], cntr[...])
    h1_out[...] = jnp.maximum(
        _dot_t(mean, wl[...]) + _dot_t(x[...], wr[...]) + b[...], 0.0)


def _tc2_body(sumr, cntr, h1, wl, wr, b, wi, bi_bh, wo, bo, sig_out, hid_out):
    mean = _mean(sumr[...], cntr[...])
    h2 = _dot_t(mean, wl[...]) + _dot_t(h1[...], wr[...]) + b[...]
    hidden = jnp.tanh(_dot_t(h2, wi[...]) + bi_bh[...])
    hid_out[...] = hidden
    sig_out[...] = jax.nn.sigmoid(_dot_t(hidden, wo[...]) + bo[...])


_row_spec = pl.BlockSpec((R, D), lambda i: (i, 0))
_sum_spec = pl.BlockSpec((2, R, D), lambda i: (0, i, 0))
_cnt_spec = pl.BlockSpec((R, NW), lambda i: (i, 0))
_w_spec = pl.BlockSpec((H, D), lambda i: (0, 0))
_b_spec = pl.BlockSpec((1, H), lambda i: (0, 0))


def kernel(x, edge_index, W_l1, W_r1, b1, W_l2, W_r2, b2, Wi, bi, Wh, bh, Wo, bo):
    src2 = edge_index[0].reshape(NW, EPT)
    dst2 = edge_index[1].reshape(NW, EPT)
    src = src2[:, :NCHUNK * CH].reshape(NW * NCHUNK, 1, CH)
    dst = dst2[:, :NCHUNK * CH].reshape(NW * NCHUNK, 1, CH)
    src_t = src2[:, NCHUNK * CH:].reshape(NW, 1, CHT)
    dst_t = dst2[:, NCHUNK * CH:].reshape(NW, 1, CHT)
    zf = jnp.zeros((CH, D), jnp.float32)
    zc = jnp.zeros((N_PAD,), jnp.float32)

    sum1, cnt = _sc_agg_counts(x, src, dst, src_t, dst_t, zf, zc)
    sum1 = sum1.reshape(NC, N_PAD, D)
    cnt = cnt.T

    b1r = b1.reshape(1, H)
    h1 = pl.pallas_call(
        _tc1_body,
        grid=(NBLK,),
        in_specs=[_sum_spec, _cnt_spec, _row_spec, _w_spec, _w_spec, _b_spec],
        out_specs=_row_spec,
        out_shape=jax.ShapeDtypeStruct((N, H), jnp.float32),
    )(sum1, cnt, x, W_l1, W_r1, b1r)

    (sum2,) = _sc_agg(h1, src, dst, src_t, dst_t, zf, zc)
    sum2 = sum2.reshape(NC, N_PAD, D)

    sig, hidden = pl.pallas_call(
        _tc2_body,
        grid=(NBLK,),
        in_specs=[_sum_spec, _cnt_spec, _row_spec, _w_spec, _w_spec, _b_spec,
                  _w_spec, _b_spec, _w_spec, _b_spec],
        out_specs=[_row_spec, _row_spec],
        out_shape=[jax.ShapeDtypeStruct((N, H), jnp.float32),
                   jax.ShapeDtypeStruct((N, H), jnp.float32)],
    )(sum2, cnt, h1, W_l2, W_r2, b2.reshape(1, H), Wi,
      (bi + bh).reshape(1, H), Wo, bo.reshape(1, H))

    return (sig, hidden)
